# baseline (speedup 1.0000x reference)
import functools

import jax
import jax.numpy as jnp
from jax import lax
from jax.experimental import pallas as pl
from jax.experimental.pallas import tpu as pltpu
from jax.experimental.pallas import tpu_sc as plsc

N = 10000
D = 128
E = 320000
NC = 2
NS = 16
NW = NC * NS
EPW = E // NW
CHUNK = 128
NCHUNK = 80
PAD = NCHUNK * CHUNK - EPW
BLK = 4
NBLK = NCHUNK // BLK
NPAIR = NBLK // 2
NACC = N + 8
RPT = 624
TAIL = N - NS * RPT
LANES = 16

_mesh = plsc.VectorSubcoreMesh(core_axis_name="c", subcore_axis_name="s")


@functools.partial(
    pl.kernel,
    mesh=_mesh,
    out_type=jax.ShapeDtypeStruct((NC, N, D), jnp.float32),
    scratch_types=[
        pltpu.VMEM_SHARED((NACC, D), jnp.float32),
        pltpu.VMEM((2, BLK, 2, CHUNK), jnp.int32),
        pltpu.VMEM((2, CHUNK, D), jnp.float32),
        pltpu.SemaphoreType.DMA,
        pltpu.SemaphoreType.DMA,
        pltpu.SemaphoreType.DMA,
        pltpu.SemaphoreType.DMA,
        pltpu.SemaphoreType.DMA,
    ],
)
def _sc_gather_scatter(x_hbm, e_hbm, part_hbm,
                       acc, ebuf, rows_v, zsem, isem0, isem1, gsem0, gsem1):
    c = lax.axis_index("c")
    s = lax.axis_index("s")
    w = c * NS + s
    isems = (isem0, isem1)
    gsems = (gsem0, gsem1)

    pltpu.async_copy(e_hbm.at[w].at[pl.ds(0, BLK)], ebuf.at[0], isem0)
    pltpu.async_copy(e_hbm.at[w].at[pl.ds(BLK, BLK)], ebuf.at[1], isem1)

    zero = jnp.zeros((LANES,), jnp.float32)

    @pl.loop(0, CHUNK)
    def _(i):
        for j in range(D // LANES):
            rows_v[0, i, pl.ds(j * LANES, LANES)] = zero

    zcps = []
    for t in range(4):
        zcps.append(pltpu.async_copy(
            rows_v.at[0].at[pl.ds(0, CHUNK)],
            acc.at[pl.ds(s * RPT + t * CHUNK, CHUNK)], zsem))
    zcps.append(pltpu.async_copy(
        rows_v.at[0].at[pl.ds(0, RPT - 4 * CHUNK)],
        acc.at[pl.ds(s * RPT + 4 * CHUNK, RPT - 4 * CHUNK)], zsem))

    @pl.when(s == NS - 1)
    def _():
        pltpu.async_copy(rows_v.at[0].at[pl.ds(0, TAIL)],
                         acc.at[pl.ds(NS * RPT, TAIL)], zsem).wait()

    for cp in zcps:
        cp.wait()

    pltpu.make_async_copy(e_hbm.at[w].at[pl.ds(0, BLK)], ebuf.at[0],
                          isem0).wait()
    for b in range(2):
        pltpu.async_copy(x_hbm.at[ebuf.at[0].at[b].at[0]], rows_v.at[b],
                         gsems[b])

    plsc.subcore_barrier()

    @pl.loop(0, NPAIR)
    def _(kp):
        for m in range(2 * BLK):
            p_cur = m // BLK
            r_cur = m % BLK
            b = m % 2
            p_nxt = ((m + 2) // BLK) % 2
            r_nxt = (m + 2) % BLK

            pltpu.make_async_copy(x_hbm.at[ebuf.at[p_cur].at[r_cur].at[0]],
                                  rows_v.at[b], gsems[b]).wait()
            pltpu.sync_copy(rows_v.at[b],
                            acc.at[ebuf.at[p_cur].at[r_cur].at[1]], add=True)

            if m == BLK - 2:
                pltpu.make_async_copy(e_hbm.at[w].at[pl.ds(0, BLK)],
                                      ebuf.at[1], isem1).wait()

            if m < 2 * BLK - 2:
                pltpu.async_copy(x_hbm.at[ebuf.at[p_nxt].at[r_nxt].at[0]],
                                 rows_v.at[b], gsems[b])
            else:
                @pl.when(kp < NPAIR - 1)
                def _():
                    if m == 2 * BLK - 2:
                        pltpu.make_async_copy(e_hbm.at[w].at[pl.ds(0, BLK)],
                                              ebuf.at[0], isem0).wait()
                    pltpu.async_copy(x_hbm.at[ebuf.at[p_nxt].at[r_nxt].at[0]],
                                     rows_v.at[b], gsems[b])

            if m == BLK - 1:
                @pl.when(kp < NPAIR - 1)
                def _():
                    pltpu.async_copy(
                        e_hbm.at[w].at[pl.ds((2 * kp + 2) * BLK, BLK)],
                        ebuf.at[0], isem0)
            if m == 2 * BLK - 1:
                @pl.when(kp < NPAIR - 1)
                def _():
                    pltpu.async_copy(
                        e_hbm.at[w].at[pl.ds((2 * kp + 3) * BLK, BLK)],
                        ebuf.at[1], isem1)

    plsc.subcore_barrier()

    pltpu.sync_copy(
        acc.at[pl.ds(s * RPT, RPT)],
        part_hbm.at[c].at[pl.ds(s * RPT, RPT)],
    )

    @pl.when(s == NS - 1)
    def _():
        pltpu.sync_copy(
            acc.at[pl.ds(NS * RPT, TAIL)],
            part_hbm.at[c].at[pl.ds(NS * RPT, TAIL)],
        )


def _add_partials(p_ref, o_ref):
    o_ref[...] = p_ref[0] + p_ref[1]


def kernel(x, edge_index):
    ei = edge_index.astype(jnp.int32)
    src = jnp.pad(ei[0].reshape(NW, EPW), ((0, 0), (0, PAD)),
                  constant_values=0)
    dst = jnp.pad(ei[1].reshape(NW, EPW), ((0, 0), (0, PAD)),
                  constant_values=N)
    e = jnp.stack([src.reshape(NW, NCHUNK, CHUNK),
                   dst.reshape(NW, NCHUNK, CHUNK)], axis=2)
    part = _sc_gather_scatter(x, e)
    out = pl.pallas_call(
        _add_partials,
        grid=(10,),
        in_specs=[pl.BlockSpec((NC, N // 10, D), lambda i: (0, i, 0))],
        out_specs=pl.BlockSpec((N // 10, D), lambda i: (i, 0)),
        out_shape=jax.ShapeDtypeStruct((N, D), jnp.float32),
    )(part)
    return out
